# baseline (speedup 1.0000x reference)
import functools

import jax
import jax.numpy as jnp
from jax.experimental import pallas as pl
from jax.experimental.pallas import tpu as pltpu

N = 10000
E_REAL = 330000
EB = 4096
N_EB = 81
E_PAD = N_EB * EB
NB = 1000
N_NB = N // NB
NCHUNK = 2000
N_CHUNKS = N // NCHUNK


def _mm_kernel(pre_act, x_r, w_r, as_r, ad_r, b_r, h_r, os_r, od_r):
    x = x_r[...]
    if pre_act:
        x = jnp.maximum(x + b_r[...], 0.0)
    h = jnp.dot(x, w_r[...], preferred_element_type=jnp.float32)
    h_r[...] = h
    os_r[...] = jnp.dot(h, as_r[...], preferred_element_type=jnp.float32)
    od_r[...] = jnp.dot(h, ad_r[...], preferred_element_type=jnp.float32)


def _edge_e_kernel(src_r, dst_r, as_r, ad_r, e_r, m_r):
    step = pl.program_id(0)

    @pl.when(step == 0)
    def _():
        m_r[0, 0] = -jnp.inf

    def body(i, mx):
        s = src_r[0, 0, i]
        d = dst_r[0, 0, i]
        v = as_r[s] + ad_r[d]
        v = jnp.where(v >= 0.0, v, 0.2 * v)
        e_r[0, 0, i] = v
        return jnp.maximum(mx, v)

    mx = jax.lax.fori_loop(0, EB, body, -jnp.inf)
    m_r[0, 0] = jnp.maximum(m_r[0, 0], mx)


def _g_kernel(e_r, m_r, g_r):
    step = pl.program_id(0)
    idx = jax.lax.broadcasted_iota(jnp.int32, (1, 1, EB), 2) + step * EB
    g_r[...] = jnp.where(idx < E_REAL, jnp.exp(e_r[...] - m_r[0, 0]), 0.0)


def _denom_kernel(dst_r, g_r, den_r):
    step = pl.program_id(0)

    @pl.when(step == 0)
    def _():
        def zb(i, c):
            den_r[i] = 0.0
            return c
        jax.lax.fori_loop(0, N, zb, 0)

    def body(i, c):
        d = dst_r[0, 0, i]
        den_r[d] = den_r[d] + g_r[0, 0, i]
        return c

    jax.lax.fori_loop(0, EB, body, 0)


def _agg_kernel(src_r, dst_r, g_r, den_r, h_r, out_r):
    chunk = pl.program_id(0)
    step = pl.program_id(1)
    lo = chunk * NCHUNK

    @pl.when(step == 0)
    def _():
        out_r[...] = jnp.zeros_like(out_r)

    def body(i, c):
        d = dst_r[0, 0, i]
        dd = d - lo

        @pl.when((dd >= 0) & (dd < NCHUNK))
        def _():
            s = src_r[0, 0, i]
            a = g_r[0, 0, i] / (den_r[d] + 1e-16)
            row = h_r[pl.ds(s, 1), :]
            out_r[pl.ds(dd, 1), :] = out_r[pl.ds(dd, 1), :] + a * row

        return c

    jax.lax.fori_loop(0, EB, body, 0)


def _fc_kernel(x_r, b2_r, wfc_r, bfc_r, out_r):
    z = jnp.maximum(x_r[...] + b2_r[...], 0.0)
    logits = jnp.dot(z, wfc_r[...], preferred_element_type=jnp.float32) + bfc_r[...]
    mx = jnp.max(logits, axis=1, keepdims=True)
    ex = jnp.exp(logits - mx)
    out_r[...] = ex / jnp.sum(ex, axis=1, keepdims=True)


def _smem(shape, imap):
    return pl.BlockSpec(shape, imap, memory_space=pltpu.SMEM)


def _gat_layer(z, src2, dst2, W, a_s, a_d, b_pre, pre_act):
    din = z.shape[1]
    dout = W.shape[1]
    b_in = b_pre.reshape(1, din) if pre_act else jnp.zeros((1, din), jnp.float32)

    h, als, ald = pl.pallas_call(
        functools.partial(_mm_kernel, pre_act),
        grid=(N_NB,),
        in_specs=[
            pl.BlockSpec((NB, din), lambda i: (i, 0)),
            pl.BlockSpec((din, dout), lambda i: (0, 0)),
            pl.BlockSpec((dout, 1), lambda i: (0, 0)),
            pl.BlockSpec((dout, 1), lambda i: (0, 0)),
            pl.BlockSpec((1, din), lambda i: (0, 0)),
        ],
        out_specs=[
            pl.BlockSpec((NB, dout), lambda i: (i, 0)),
            pl.BlockSpec((NB, 1), lambda i: (i, 0)),
            pl.BlockSpec((NB, 1), lambda i: (i, 0)),
        ],
        out_shape=[
            jax.ShapeDtypeStruct((N, dout), jnp.float32),
            jax.ShapeDtypeStruct((N, 1), jnp.float32),
            jax.ShapeDtypeStruct((N, 1), jnp.float32),
        ],
    )(z, W, a_s.reshape(dout, 1), a_d.reshape(dout, 1), b_in)

    als = als.reshape(N)
    ald = ald.reshape(N)

    e, m = pl.pallas_call(
        _edge_e_kernel,
        grid=(N_EB,),
        in_specs=[
            _smem((1, 1, EB), lambda i: (i, 0, 0)),
            _smem((1, 1, EB), lambda i: (i, 0, 0)),
            _smem((N,), lambda i: (0,)),
            _smem((N,), lambda i: (0,)),
        ],
        out_specs=[
            _smem((1, 1, EB), lambda i: (i, 0, 0)),
            _smem((1, 1), lambda i: (0, 0)),
        ],
        out_shape=[
            jax.ShapeDtypeStruct((N_EB, 1, EB), jnp.float32),
            jax.ShapeDtypeStruct((1, 1), jnp.float32),
        ],
    )(src2, dst2, als, ald)

    g = pl.pallas_call(
        _g_kernel,
        grid=(N_EB,),
        in_specs=[
            pl.BlockSpec((1, 1, EB), lambda i: (i, 0, 0)),
            _smem((1, 1), lambda i: (0, 0)),
        ],
        out_specs=pl.BlockSpec((1, 1, EB), lambda i: (i, 0, 0)),
        out_shape=jax.ShapeDtypeStruct((N_EB, 1, EB), jnp.float32),
    )(e, m)

    den = pl.pallas_call(
        _denom_kernel,
        grid=(N_EB,),
        in_specs=[
            _smem((1, 1, EB), lambda i: (i, 0, 0)),
            _smem((1, 1, EB), lambda i: (i, 0, 0)),
        ],
        out_specs=_smem((N,), lambda i: (0,)),
        out_shape=jax.ShapeDtypeStruct((N,), jnp.float32),
    )(dst2, g)

    out = pl.pallas_call(
        _agg_kernel,
        grid=(N_CHUNKS, N_EB),
        in_specs=[
            _smem((1, 1, EB), lambda c, i: (i, 0, 0)),
            _smem((1, 1, EB), lambda c, i: (i, 0, 0)),
            _smem((1, 1, EB), lambda c, i: (i, 0, 0)),
            _smem((N,), lambda c, i: (0,)),
            pl.BlockSpec((N, dout), lambda c, i: (0, 0)),
        ],
        out_specs=pl.BlockSpec((NCHUNK, dout), lambda c, i: (c, 0)),
        out_shape=jax.ShapeDtypeStruct((N, dout), jnp.float32),
    )(src2, dst2, g, den, h)

    return out


def kernel(x, edge_index, W1, a1_src, a1_dst, b1, W2, a2_src, a2_dst, b2, Wfc, bfc):
    loop = jnp.arange(N, dtype=edge_index.dtype)
    src = jnp.concatenate([edge_index[0], loop])
    dst = jnp.concatenate([edge_index[1], loop])
    pad = E_PAD - E_REAL
    src2 = jnp.pad(src, (0, pad)).reshape(N_EB, 1, EB)
    dst2 = jnp.pad(dst, (0, pad)).reshape(N_EB, 1, EB)

    s1 = _gat_layer(x, src2, dst2, W1, a1_src, a1_dst, b1, pre_act=False)
    s2 = _gat_layer(s1, src2, dst2, W2, a2_src, a2_dst, b1, pre_act=True)

    out = pl.pallas_call(
        _fc_kernel,
        grid=(N_NB,),
        in_specs=[
            pl.BlockSpec((NB, 1024), lambda i: (i, 0)),
            pl.BlockSpec((1, 1024), lambda i: (0, 0)),
            pl.BlockSpec((1024, 10), lambda i: (0, 0)),
            pl.BlockSpec((1, 10), lambda i: (0, 0)),
        ],
        out_specs=pl.BlockSpec((NB, 10), lambda i: (i, 0)),
        out_shape=jax.ShapeDtypeStruct((N, 10), jnp.float32),
    )(s2, b2.reshape(1, 1024), Wfc, bfc.reshape(1, 10))
    return out


# speedup vs baseline: 1.7436x; 1.7436x over previous
import functools

import jax
import jax.numpy as jnp
from jax.experimental import pallas as pl
from jax.experimental.pallas import tpu as pltpu

N = 10000
E_REAL = 330000
EB = 4096
N_EB = 81
E_PAD = N_EB * EB
NB = 1000
N_NB = N // NB
NCHUNK = 2000
N_CHUNKS = N // NCHUNK


def _mm_kernel(pre_act, x_r, w_r, as_r, ad_r, b_r, h_r, os_r, od_r):
    x = x_r[...]
    if pre_act:
        x = jnp.maximum(x + b_r[...], 0.0)
    h = jnp.dot(x, w_r[...], preferred_element_type=jnp.float32)
    h_r[...] = h
    os_r[...] = jnp.dot(h, as_r[...], preferred_element_type=jnp.float32)
    od_r[...] = jnp.dot(h, ad_r[...], preferred_element_type=jnp.float32)


def _edge_e_kernel(src_r, dst_r, as_r, ad_r, e_r, m_r):
    step = pl.program_id(0)

    @pl.when(step == 0)
    def _():
        m_r[0, 0] = -jnp.inf

    def body(i, mx):
        s = src_r[0, 0, i]
        d = dst_r[0, 0, i]
        v = as_r[s] + ad_r[d]
        v = jnp.where(v >= 0.0, v, 0.2 * v)
        e_r[0, 0, i] = v
        return jnp.maximum(mx, v)

    mx = jax.lax.fori_loop(0, EB, body, -jnp.inf)
    m_r[0, 0] = jnp.maximum(m_r[0, 0], mx)


def _g_kernel(e_r, m_r, g_r):
    step = pl.program_id(0)
    idx = jax.lax.broadcasted_iota(jnp.int32, (1, 1, EB), 2) + step * EB
    g_r[...] = jnp.where(idx < E_REAL, jnp.exp(e_r[...] - m_r[0, 0]), 0.0)


def _denom_kernel(dst_r, g_r, den_r):
    step = pl.program_id(0)

    @pl.when(step == 0)
    def _():
        def zb(i, c):
            den_r[i] = 0.0
            return c
        jax.lax.fori_loop(0, N, zb, 0)

    def body(i, c):
        d = dst_r[0, 0, i]
        den_r[d] = den_r[d] + g_r[0, 0, i]
        return c

    jax.lax.fori_loop(0, EB, body, 0)


def _agg_kernel(flag_r, src_r, dst_r, g_r, den_r, h_r, out_r):
    chunk = pl.program_id(0)
    step = pl.program_id(1)
    lo = chunk * NCHUNK

    @pl.when(step == 0)
    def _():
        out_r[...] = jnp.zeros_like(out_r)

    @pl.when(flag_r[0, 0, 0, 0] != 0)
    def _():
        def body(i, c):
            d = dst_r[0, 0, i]
            dd = d - lo

            @pl.when((dd >= 0) & (dd < NCHUNK))
            def _():
                s = src_r[0, 0, i]
                a = g_r[0, 0, i] / (den_r[d] + 1e-16)
                row = h_r[pl.ds(s, 1), :]
                out_r[pl.ds(dd, 1), :] = out_r[pl.ds(dd, 1), :] + a * row

            return c

        jax.lax.fori_loop(0, EB, body, 0)


def _fc_kernel(x_r, b2_r, wfc_r, bfc_r, out_r):
    z = jnp.maximum(x_r[...] + b2_r[...], 0.0)
    logits = jnp.dot(z, wfc_r[...], preferred_element_type=jnp.float32) + bfc_r[...]
    mx = jnp.max(logits, axis=1, keepdims=True)
    ex = jnp.exp(logits - mx)
    out_r[...] = ex / jnp.sum(ex, axis=1, keepdims=True)


def _smem(shape, imap):
    return pl.BlockSpec(shape, imap, memory_space=pltpu.SMEM)


def _gat_layer(z, src2, dst2, flags, W, a_s, a_d, b_pre, pre_act):
    din = z.shape[1]
    dout = W.shape[1]
    b_in = b_pre.reshape(1, din) if pre_act else jnp.zeros((1, din), jnp.float32)

    h, als, ald = pl.pallas_call(
        functools.partial(_mm_kernel, pre_act),
        grid=(N_NB,),
        in_specs=[
            pl.BlockSpec((NB, din), lambda i: (i, 0)),
            pl.BlockSpec((din, dout), lambda i: (0, 0)),
            pl.BlockSpec((dout, 1), lambda i: (0, 0)),
            pl.BlockSpec((dout, 1), lambda i: (0, 0)),
            pl.BlockSpec((1, din), lambda i: (0, 0)),
        ],
        out_specs=[
            pl.BlockSpec((NB, dout), lambda i: (i, 0)),
            pl.BlockSpec((NB, 1), lambda i: (i, 0)),
            pl.BlockSpec((NB, 1), lambda i: (i, 0)),
        ],
        out_shape=[
            jax.ShapeDtypeStruct((N, dout), jnp.float32),
            jax.ShapeDtypeStruct((N, 1), jnp.float32),
            jax.ShapeDtypeStruct((N, 1), jnp.float32),
        ],
    )(z, W, a_s.reshape(dout, 1), a_d.reshape(dout, 1), b_in)

    als = als.reshape(N)
    ald = ald.reshape(N)

    e, m = pl.pallas_call(
        _edge_e_kernel,
        grid=(N_EB,),
        in_specs=[
            _smem((1, 1, EB), lambda i: (i, 0, 0)),
            _smem((1, 1, EB), lambda i: (i, 0, 0)),
            _smem((N,), lambda i: (0,)),
            _smem((N,), lambda i: (0,)),
        ],
        out_specs=[
            _smem((1, 1, EB), lambda i: (i, 0, 0)),
            _smem((1, 1), lambda i: (0, 0)),
        ],
        out_shape=[
            jax.ShapeDtypeStruct((N_EB, 1, EB), jnp.float32),
            jax.ShapeDtypeStruct((1, 1), jnp.float32),
        ],
    )(src2, dst2, als, ald)

    g = pl.pallas_call(
        _g_kernel,
        grid=(N_EB,),
        in_specs=[
            pl.BlockSpec((1, 1, EB), lambda i: (i, 0, 0)),
            _smem((1, 1), lambda i: (0, 0)),
        ],
        out_specs=pl.BlockSpec((1, 1, EB), lambda i: (i, 0, 0)),
        out_shape=jax.ShapeDtypeStruct((N_EB, 1, EB), jnp.float32),
    )(e, m)

    den = pl.pallas_call(
        _denom_kernel,
        grid=(N_EB,),
        in_specs=[
            _smem((1, 1, EB), lambda i: (i, 0, 0)),
            _smem((1, 1, EB), lambda i: (i, 0, 0)),
        ],
        out_specs=_smem((N,), lambda i: (0,)),
        out_shape=jax.ShapeDtypeStruct((N,), jnp.float32),
    )(dst2, g)

    out = pl.pallas_call(
        _agg_kernel,
        grid=(N_CHUNKS, N_EB),
        in_specs=[
            _smem((1, 1, 1, 1), lambda c, i: (c, i, 0, 0)),
            _smem((1, 1, EB), lambda c, i: (i, 0, 0)),
            _smem((1, 1, EB), lambda c, i: (i, 0, 0)),
            _smem((1, 1, EB), lambda c, i: (i, 0, 0)),
            _smem((N,), lambda c, i: (0,)),
            pl.BlockSpec((N, dout), lambda c, i: (0, 0)),
        ],
        out_specs=pl.BlockSpec((NCHUNK, dout), lambda c, i: (c, 0)),
        out_shape=jax.ShapeDtypeStruct((N, dout), jnp.float32),
    )(flags, src2, dst2, g, den, h)

    return out


def kernel(x, edge_index, W1, a1_src, a1_dst, b1, W2, a2_src, a2_dst, b2, Wfc, bfc):
    loop = jnp.arange(N, dtype=edge_index.dtype)
    src = jnp.concatenate([edge_index[0], loop])
    dst = jnp.concatenate([edge_index[1], loop])
    order = jnp.argsort(dst)
    src = src[order]
    dst = dst[order]
    pad = E_PAD - E_REAL
    src2 = jnp.pad(src, (0, pad)).reshape(N_EB, 1, EB)
    dst2 = jnp.pad(dst, (0, pad)).reshape(N_EB, 1, EB)
    bmin = dst2.min(axis=(1, 2))
    bmax = dst2.max(axis=(1, 2))
    clo = jnp.arange(N_CHUNKS, dtype=jnp.int32)[:, None] * NCHUNK
    flags = ((bmax[None, :] >= clo) & (bmin[None, :] < clo + NCHUNK)).astype(jnp.int32)
    flags = flags.reshape(N_CHUNKS, N_EB, 1, 1)

    s1 = _gat_layer(x, src2, dst2, flags, W1, a1_src, a1_dst, b1, pre_act=False)
    s2 = _gat_layer(s1, src2, dst2, flags, W2, a2_src, a2_dst, b1, pre_act=True)

    out = pl.pallas_call(
        _fc_kernel,
        grid=(N_NB,),
        in_specs=[
            pl.BlockSpec((NB, 1024), lambda i: (i, 0)),
            pl.BlockSpec((1, 1024), lambda i: (0, 0)),
            pl.BlockSpec((1024, 10), lambda i: (0, 0)),
            pl.BlockSpec((1, 10), lambda i: (0, 0)),
        ],
        out_specs=pl.BlockSpec((NB, 10), lambda i: (i, 0)),
        out_shape=jax.ShapeDtypeStruct((N, 10), jnp.float32),
    )(s2, b2.reshape(1, 1024), Wfc, bfc.reshape(1, 10))
    return out
